# baseline (speedup 1.0000x reference)
import functools

import jax
import jax.numpy as jnp
import numpy as np
from jax.experimental import pallas as pl
from jax.experimental.pallas import tpu as pltpu
from jax.sharding import Mesh, PartitionSpec as P

N = 10000
D = 128
H1 = 64
H2 = 32

BM = 200


def _k1_small(x_ref, w1_ref, wa1_ref, wa2_ref, wa3_ref,
              xw1_ref, mua_ref, logvara_ref):
    x = x_ref[...]
    xw1_ref[...] = jnp.dot(x, w1_ref[...], preferred_element_type=jnp.float32)
    ha1 = jnp.tanh(jax.lax.dot_general(
        x, wa1_ref[...], (((0,), (0,)), ((), ())),
        preferred_element_type=jnp.float32))
    mua_ref[...] = jnp.dot(ha1, wa2_ref[...], preferred_element_type=jnp.float32)
    logvara_ref[...] = jnp.dot(ha1, wa3_ref[...], preferred_element_type=jnp.float32)


def _k2_pass1(adj_ref, xw1_ref, w23_ref, hw_ref):
    h1 = jnp.maximum(
        jnp.dot(adj_ref[...], xw1_ref[...], preferred_element_type=jnp.float32),
        0.0)
    hw_ref[...] = jnp.dot(h1, w23_ref[...], preferred_element_type=jnp.float32)


def _k3_pass2(adj_ref, hw_ref, mua_ref, mu_ref, logvar_ref, feat_ref):
    ml = jnp.dot(adj_ref[...], hw_ref[...], preferred_element_type=jnp.float32)
    mu = ml[:, :H2]
    mu_ref[...] = mu
    logvar_ref[...] = ml[:, H2:]
    feat_ref[...] = jax.lax.dot_general(
        mu, mua_ref[...], (((1,), (1,)), ((), ())),
        preferred_element_type=jnp.float32)


def _k4_decoder(zi_ref, zj_ref, out_ref):
    out_ref[...] = jax.lax.dot_general(
        zi_ref[...], zj_ref[...], (((1,), (1,)), ((), ())),
        preferred_element_type=jnp.float32)


def _pipeline(x, adj_loc, W1, W2, W3, Wa1, Wa2, Wa3, axis_name):
    f32 = jnp.float32
    m_loc = adj_loc.shape[0]
    nstrip = m_loc // BM

    xw1, mu_a, logvar_a = pl.pallas_call(
        _k1_small,
        out_shape=(
            jax.ShapeDtypeStruct((N, H1), f32),
            jax.ShapeDtypeStruct((D, H2), f32),
            jax.ShapeDtypeStruct((D, H2), f32),
        ),
    )(x, W1, Wa1, Wa2, Wa3)

    w23 = jnp.concatenate([W2, W3], axis=1)

    hw_loc = pl.pallas_call(
        _k2_pass1,
        grid=(nstrip,),
        in_specs=[
            pl.BlockSpec((BM, N), lambda i: (i, 0)),
            pl.BlockSpec((N, H1), lambda i: (0, 0)),
            pl.BlockSpec((H1, 2 * H2), lambda i: (0, 0)),
        ],
        out_specs=pl.BlockSpec((BM, 2 * H2), lambda i: (i, 0)),
        out_shape=jax.ShapeDtypeStruct((m_loc, 2 * H2), f32),
        compiler_params=pltpu.CompilerParams(
            dimension_semantics=("arbitrary",)),
    )(adj_loc, xw1, w23)

    if axis_name is not None:
        hw = jax.lax.all_gather(hw_loc, axis_name, axis=0, tiled=True)
    else:
        hw = hw_loc

    mu_loc, logvar_loc, feat_loc = pl.pallas_call(
        _k3_pass2,
        grid=(nstrip,),
        in_specs=[
            pl.BlockSpec((BM, N), lambda i: (i, 0)),
            pl.BlockSpec((N, 2 * H2), lambda i: (0, 0)),
            pl.BlockSpec((D, H2), lambda i: (0, 0)),
        ],
        out_specs=(
            pl.BlockSpec((BM, H2), lambda i: (i, 0)),
            pl.BlockSpec((BM, H2), lambda i: (i, 0)),
            pl.BlockSpec((BM, D), lambda i: (i, 0)),
        ),
        out_shape=(
            jax.ShapeDtypeStruct((m_loc, H2), f32),
            jax.ShapeDtypeStruct((m_loc, H2), f32),
            jax.ShapeDtypeStruct((m_loc, D), f32),
        ),
        compiler_params=pltpu.CompilerParams(
            dimension_semantics=("arbitrary",)),
    )(adj_loc, hw, mu_a)

    if axis_name is not None:
        mu = jax.lax.all_gather(mu_loc, axis_name, axis=0, tiled=True)
    else:
        mu = mu_loc

    adjrec_loc = pl.pallas_call(
        _k4_decoder,
        grid=(nstrip,),
        in_specs=[
            pl.BlockSpec((BM, H2), lambda i: (i, 0)),
            pl.BlockSpec((N, H2), lambda i: (0, 0)),
        ],
        out_specs=pl.BlockSpec((BM, N), lambda i: (i, 0)),
        out_shape=jax.ShapeDtypeStruct((m_loc, N), f32),
        compiler_params=pltpu.CompilerParams(
            dimension_semantics=("arbitrary",)),
    )(mu_loc, mu)

    return adjrec_loc, feat_loc, mu_loc, logvar_loc, mu_a, logvar_a


def kernel(x, adj, W1, W2, W3, Wa1, Wa2, Wa3):
    devs = jax.devices()
    if len(devs) >= 2:
        mesh = Mesh(np.array(devs[:2]), ("x",))
        f = functools.partial(_pipeline, axis_name="x")
        shmap = jax.shard_map(
            f, mesh=mesh,
            in_specs=(P(None, None), P("x", None), P(None, None),
                      P(None, None), P(None, None), P(None, None),
                      P(None, None), P(None, None)),
            out_specs=(P("x", None), P("x", None), P("x", None),
                       P("x", None), P(None, None), P(None, None)),
            check_vma=False,
        )
        return shmap(x, adj, W1, W2, W3, Wa1, Wa2, Wa3)
    return _pipeline(x, adj, W1, W2, W3, Wa1, Wa2, Wa3, axis_name=None)


# speedup vs baseline: 1.3629x; 1.3629x over previous
import functools

import jax
import jax.numpy as jnp
import numpy as np
from jax.experimental import pallas as pl
from jax.experimental.pallas import tpu as pltpu
from jax.sharding import Mesh, PartitionSpec as P

N = 10000
D = 128
H1 = 64
H2 = 32

BM = 400
BD = 200


def _k1_small(x_ref, w1_ref, wa1_ref, wa2_ref, wa3_ref,
              xw1_ref, mua_ref, logvara_ref):
    x = x_ref[...]
    xw1_ref[...] = jnp.dot(x, w1_ref[...], preferred_element_type=jnp.float32)
    ha1 = jnp.tanh(jax.lax.dot_general(
        x, wa1_ref[...], (((0,), (0,)), ((), ())),
        preferred_element_type=jnp.float32))
    mua_ref[...] = jnp.dot(ha1, wa2_ref[...], preferred_element_type=jnp.float32)
    logvara_ref[...] = jnp.dot(ha1, wa3_ref[...], preferred_element_type=jnp.float32)


def _k2_pass1(adj_ref, xw1_ref, w23_ref, hw_ref):
    h1 = jnp.maximum(
        jnp.dot(adj_ref[...], xw1_ref[...], preferred_element_type=jnp.float32),
        0.0)
    hw_ref[...] = jnp.dot(h1, w23_ref[...], preferred_element_type=jnp.float32)


def _k3_pass2(adj_ref, hw_ref, mua_ref, mu_ref, logvar_ref, feat_ref):
    ml = jnp.dot(adj_ref[...], hw_ref[...], preferred_element_type=jnp.float32)
    mu = ml[:, :H2]
    mu_ref[...] = mu
    logvar_ref[...] = ml[:, H2:]
    feat_ref[...] = jax.lax.dot_general(
        mu, mua_ref[...], (((1,), (1,)), ((), ())),
        preferred_element_type=jnp.float32)


def _k4_decoder(zi_ref, zj_ref, out_ref):
    out_ref[...] = jax.lax.dot_general(
        zi_ref[...], zj_ref[...], (((1,), (1,)), ((), ())),
        preferred_element_type=jnp.float32)


def _pipeline(x, adj, W1, W2, W3, Wa1, Wa2, Wa3, nshard, dev_index):
    f32 = jnp.float32
    nstrip = N // BM

    xw1, mu_a, logvar_a = pl.pallas_call(
        _k1_small,
        out_shape=(
            jax.ShapeDtypeStruct((N, H1), f32),
            jax.ShapeDtypeStruct((D, H2), f32),
            jax.ShapeDtypeStruct((D, H2), f32),
        ),
    )(x, W1, Wa1, Wa2, Wa3)

    w23 = jnp.concatenate([W2, W3], axis=1)

    hw = pl.pallas_call(
        _k2_pass1,
        grid=(nstrip,),
        in_specs=[
            pl.BlockSpec((BM, N), lambda i: (i, 0)),
            pl.BlockSpec((N, H1), lambda i: (0, 0)),
            pl.BlockSpec((H1, 2 * H2), lambda i: (0, 0)),
        ],
        out_specs=pl.BlockSpec((BM, 2 * H2), lambda i: (i, 0)),
        out_shape=jax.ShapeDtypeStruct((N, 2 * H2), f32),
        compiler_params=pltpu.CompilerParams(
            dimension_semantics=("arbitrary",)),
    )(adj, xw1, w23)

    mu, logvar, features = pl.pallas_call(
        _k3_pass2,
        grid=(nstrip,),
        in_specs=[
            pl.BlockSpec((BM, N), lambda i: (i, 0)),
            pl.BlockSpec((N, 2 * H2), lambda i: (0, 0)),
            pl.BlockSpec((D, H2), lambda i: (0, 0)),
        ],
        out_specs=(
            pl.BlockSpec((BM, H2), lambda i: (i, 0)),
            pl.BlockSpec((BM, H2), lambda i: (i, 0)),
            pl.BlockSpec((BM, D), lambda i: (i, 0)),
        ),
        out_shape=(
            jax.ShapeDtypeStruct((N, H2), f32),
            jax.ShapeDtypeStruct((N, H2), f32),
            jax.ShapeDtypeStruct((N, D), f32),
        ),
        compiler_params=pltpu.CompilerParams(
            dimension_semantics=("arbitrary",)),
    )(adj, hw, mu_a)

    m_loc = N // nshard
    if nshard > 1:
        zi = jax.lax.dynamic_slice(mu, (dev_index * m_loc, 0), (m_loc, H2))
    else:
        zi = mu

    adjrec_loc = pl.pallas_call(
        _k4_decoder,
        grid=(m_loc // BD,),
        in_specs=[
            pl.BlockSpec((BD, H2), lambda i: (i, 0)),
            pl.BlockSpec((N, H2), lambda i: (0, 0)),
        ],
        out_specs=pl.BlockSpec((BD, N), lambda i: (i, 0)),
        out_shape=jax.ShapeDtypeStruct((m_loc, N), f32),
        compiler_params=pltpu.CompilerParams(
            dimension_semantics=("arbitrary",)),
    )(zi, mu)

    return adjrec_loc, features, mu, logvar, mu_a, logvar_a


def _sharded_body(x, adj, W1, W2, W3, Wa1, Wa2, Wa3):
    dev = jax.lax.axis_index("x")
    return _pipeline(x, adj, W1, W2, W3, Wa1, Wa2, Wa3,
                     nshard=2, dev_index=dev)


def kernel(x, adj, W1, W2, W3, Wa1, Wa2, Wa3):
    devs = jax.devices()
    if len(devs) >= 2:
        mesh = Mesh(np.array(devs[:2]), ("x",))
        rep = P(None, None)
        shmap = jax.shard_map(
            _sharded_body, mesh=mesh,
            in_specs=(rep,) * 8,
            out_specs=(P("x", None), rep, rep, rep, rep, rep),
            check_vma=False,
        )
        return shmap(x, adj, W1, W2, W3, Wa1, Wa2, Wa3)
    return _pipeline(x, adj, W1, W2, W3, Wa1, Wa2, Wa3,
                     nshard=1, dev_index=0)
